# Initial kernel scaffold; baseline (speedup 1.0000x reference)
import jax
import jax.numpy as jnp
from jax.experimental import pallas as pl


def kernel(tokens, token_labels, buffer, pointer):
    raise NotImplementedError("write your pallas kernel here")


# speedup vs baseline: 9.4195x; 9.4195x over previous
import jax
import jax.numpy as jnp
from jax import lax
from jax.experimental import pallas as pl
from jax.experimental.pallas import tpu as pltpu
from jax.experimental.pallas import tpu_sc as plsc

NUM_CLASSES = 80
DIM = 256
N_TOKENS = 131072

NC = 2
NS = 16
L = 16
NW = NC * NS

TOK_PER_W = N_TOKENS // NW
CHUNK = 128
NCHUNK = TOK_PER_W // CHUNK
GROUPS = CHUNK // L


def _mixer(table, labels3, tokens, pointer):
    mesh = plsc.VectorSubcoreMesh(core_axis_name="c", subcore_axis_name="s")

    @pl.kernel(
        out_type=jax.ShapeDtypeStruct((N_TOKENS, DIM), jnp.float32),
        mesh=mesh,
        compiler_params=pltpu.CompilerParams(needs_layout_passes=False),
        scratch_types=[
            pltpu.VMEM((NCHUNK, CHUNK), jnp.int32),
            pltpu.VMEM((NUM_CLASSES,), jnp.int32),
            pltpu.VMEM((CHUNK, DIM), jnp.float32),
            pltpu.VMEM((L, DIM), jnp.float32),
            pltpu.SemaphoreType.DMA,
        ],
    )
    def body(table_hbm, labels_hbm, tokens_hbm, ptr_hbm, out_hbm,
             lab_v, ptr_v, rows_v, tok_v, sem):
        wid = lax.axis_index("s") * NC + lax.axis_index("c")
        pltpu.sync_copy(labels_hbm.at[wid], lab_v)
        pltpu.sync_copy(ptr_hbm, ptr_v)

        def chunk_body(j, _):
            gbase = wid * TOK_PER_W + j * CHUNK
            pltpu.async_copy(table_hbm.at[lab_v.at[j]], rows_v, sem).wait()

            def group_body(g, _):
                lab16 = lab_v[j, pl.ds(g * L, L)]
                pv = plsc.load_gather(ptr_v, [lab16])
                inv = pv == 0
                cnt = jnp.sum(inv.astype(jnp.int32))

                @pl.when(cnt > 0)
                def _fixup():
                    pltpu.sync_copy(
                        tokens_hbm.at[pl.ds(gbase + g * L, L)], tok_v)
                    for t in range(L):
                        lt = g * L + t
                        lab_t = plsc.load_gather(
                            lab_v,
                            [jnp.full((L,), j, jnp.int32),
                             jnp.full((L,), lt, jnp.int32)])
                        m = plsc.load_gather(ptr_v, [lab_t]) == 0
                        for c in range(DIM // L):
                            cur = rows_v[lt, pl.ds(c * L, L)]
                            tv = tok_v[t, pl.ds(c * L, L)]
                            rows_v[lt, pl.ds(c * L, L)] = jnp.where(m, tv, cur)
                return 0

            lax.fori_loop(0, GROUPS, group_body, 0)
            pltpu.sync_copy(rows_v, out_hbm.at[pl.ds(gbase, CHUNK)])
            return 0

        lax.fori_loop(0, NCHUNK, chunk_body, 0)

    return body(table, labels3, tokens, pointer)


def kernel(tokens, token_labels, buffer, pointer):
    table = buffer[:, 0, :]
    labels3 = token_labels.astype(jnp.int32).reshape(NW, NCHUNK, CHUNK)
    return _mixer(table, labels3, tokens, pointer.astype(jnp.int32))


# speedup vs baseline: 9.7906x; 1.0394x over previous
import jax
import jax.numpy as jnp
from jax import lax
from jax.experimental import pallas as pl
from jax.experimental.pallas import tpu as pltpu
from jax.experimental.pallas import tpu_sc as plsc

NUM_CLASSES = 80
DIM = 256
N_TOKENS = 131072

NC = 2
NS = 16
L = 16
NW = NC * NS

TOK_PER_W = N_TOKENS // NW
CHUNK = 128
NCHUNK = TOK_PER_W // CHUNK
GROUPS = CHUNK // L
NBUF = 3
PRE = 2


def _mixer(table, labels3, tokens, pointer):
    mesh = plsc.VectorSubcoreMesh(core_axis_name="c", subcore_axis_name="s")

    @pl.kernel(
        out_type=jax.ShapeDtypeStruct((N_TOKENS, DIM), jnp.float32),
        mesh=mesh,
        compiler_params=pltpu.CompilerParams(needs_layout_passes=False),
        scratch_types=[
            pltpu.VMEM((NCHUNK, CHUNK), jnp.int32),
            pltpu.VMEM((NUM_CLASSES,), jnp.int32),
            pltpu.VMEM((NBUF, CHUNK, DIM), jnp.float32),
            pltpu.VMEM((L, DIM), jnp.float32),
            pltpu.SemaphoreType.DMA((NBUF,)),
            pltpu.SemaphoreType.DMA((NBUF,)),
        ],
    )
    def body(table_hbm, labels_hbm, tokens_hbm, ptr_hbm, out_hbm,
             lab_v, ptr_v, rows_v, tok_v, gsem, wsem):
        wid = lax.axis_index("s") * NC + lax.axis_index("c")
        pltpu.sync_copy(labels_hbm.at[wid], lab_v)
        pltpu.sync_copy(ptr_hbm, ptr_v)
        base = wid * TOK_PER_W

        def it(k, _):
            @pl.when(k < NCHUNK)
            def _issue():
                b = k % NBUF

                @pl.when(k >= NBUF)
                def _reclaim():
                    pltpu.make_async_copy(
                        rows_v.at[b], out_hbm.at[pl.ds(base, CHUNK)],
                        wsem.at[b]).wait()

                pltpu.make_async_copy(
                    table_hbm.at[lab_v.at[k]], rows_v.at[b],
                    gsem.at[b]).start()

            i = k - PRE

            @pl.when(i >= 0)
            def _retire():
                bi = i % NBUF
                gbase = base + i * CHUNK
                pltpu.make_async_copy(
                    table_hbm.at[lab_v.at[i]], rows_v.at[bi],
                    gsem.at[bi]).wait()

                def group_body(g, _):
                    lab16 = lab_v[i, pl.ds(g * L, L)]
                    pv = plsc.load_gather(ptr_v, [lab16])
                    inv = pv == 0
                    cnt = jnp.sum(inv.astype(jnp.int32))

                    @pl.when(cnt > 0)
                    def _fixup():
                        pltpu.sync_copy(
                            tokens_hbm.at[pl.ds(gbase + g * L, L)], tok_v)
                        for t in range(L):
                            lt = g * L + t
                            lab_t = plsc.load_gather(
                                lab_v,
                                [jnp.full((L,), i, jnp.int32),
                                 jnp.full((L,), lt, jnp.int32)])
                            m = plsc.load_gather(ptr_v, [lab_t]) == 0
                            for c in range(DIM // L):
                                cur = rows_v[bi, lt, pl.ds(c * L, L)]
                                tv = tok_v[t, pl.ds(c * L, L)]
                                rows_v[bi, lt, pl.ds(c * L, L)] = (
                                    jnp.where(m, tv, cur))
                    return 0

                lax.fori_loop(0, GROUPS, group_body, 0)
                pltpu.make_async_copy(
                    rows_v.at[bi], out_hbm.at[pl.ds(gbase, CHUNK)],
                    wsem.at[bi]).start()
            return 0

        lax.fori_loop(0, NCHUNK + PRE, it, 0)

        for b in range(NBUF):
            pltpu.make_async_copy(
                rows_v.at[b], out_hbm.at[pl.ds(base, CHUNK)],
                wsem.at[b]).wait()

    return body(table, labels3, tokens, pointer)


def kernel(tokens, token_labels, buffer, pointer):
    table = buffer[:, 0, :]
    labels3 = token_labels.astype(jnp.int32).reshape(NW, NCHUNK, CHUNK)
    return _mixer(table, labels3, tokens, pointer.astype(jnp.int32))
